# baseline (speedup 1.0000x reference)
import functools

import jax
import jax.numpy as jnp
from jax import lax
from jax.experimental import pallas as pl
from jax.experimental.pallas import tpu as pltpu
from jax.experimental.pallas import tpu_sc as plsc

_BLANK = 0
_S, _B, _V = 2048, 16, 1024
_L = 16


_BS = 256
_NB = _S // _BS


def _argmax_body(x_ref, o_ref):
    xb = x_ref[...]
    m = jnp.max(xb, axis=2, keepdims=True)
    lane = lax.broadcasted_iota(jnp.int32, xb.shape, 2)
    idx = jnp.min(jnp.where(xb == m, lane, _V), axis=2)
    o_ref[...] = idx.astype(jnp.int32).T


_argmax_call = pl.pallas_call(
    _argmax_body,
    grid=(_NB,),
    in_specs=[pl.BlockSpec((_BS, _B, _V), lambda i: (i, 0, 0))],
    out_specs=pl.BlockSpec((_B, _BS), lambda i: (0, i)),
    out_shape=jax.ShapeDtypeStruct((_B, _S), jnp.int32),
)


@functools.partial(
    pl.kernel,
    out_type=[
        jax.ShapeDtypeStruct((_B, _S), jnp.int32),
        jax.ShapeDtypeStruct((_B, _L), jnp.int32),
    ],
    mesh=plsc.VectorSubcoreMesh(
        core_axis_name="c", subcore_axis_name="s", num_cores=1),
    compiler_params=pltpu.CompilerParams(needs_layout_passes=False),
    scratch_types=[
        pltpu.VMEM((_S,), jnp.int32),
        pltpu.VMEM((_L,), jnp.int32),
        pltpu.VMEM((_S,), jnp.int32),
        pltpu.VMEM((_L,), jnp.int32),
    ],
)
def _sc_decode(ml_hbm, len_hbm, tok_hbm, cnt_hbm,
               buf_v, len_v, out_v, cnt_v):
    c = lax.axis_index("c")
    s = lax.axis_index("s")

    @pl.when(c == 0)
    def _():
        iota = lax.iota(jnp.int32, _L)
        zero_v = jnp.zeros((_L,), jnp.int32)
        neg1 = jnp.full((_L,), -1, jnp.int32)

        pltpu.sync_copy(ml_hbm.at[s], buf_v)
        pltpu.sync_copy(len_hbm, len_v)
        length = jnp.sum(jnp.where(iota == s, len_v[...], 0))

        def init_body(i, carry):
            out_v[pl.ds(i * _L, _L)] = neg1
            return carry

        lax.fori_loop(0, _S // _L, init_body, 0)

        def step(base, off_vec):
            pos = base + iota
            v = buf_v[pl.ds(base, _L)]
            pgather = plsc.load_gather(buf_v, [jnp.maximum(pos - 1, 0)])
            prev = jnp.where(pos == 0, -1, pgather)
            valid = pos < length
            keep = (v != _BLANK) & (v != prev) & valid
            dest = off_vec + plsc.cumsum(keep.astype(jnp.int32)) - 1
            plsc.store_scatter(out_v, [dest], v, mask=keep)
            return off_vec + plsc.all_reduce_population_count(keep)

        def body(i, off_vec):
            o = step(i * 4 * _L, off_vec)
            o = step(i * 4 * _L + _L, o)
            o = step(i * 4 * _L + 2 * _L, o)
            return step(i * 4 * _L + 3 * _L, o)

        nchunks4 = (length + 4 * _L - 1) // (4 * _L)
        total_vec = lax.fori_loop(0, nchunks4, body, zero_v)
        pltpu.sync_copy(out_v, tok_hbm.at[s])

        cnt_v[...] = total_vec
        pltpu.sync_copy(cnt_v, cnt_hbm.at[s])


def kernel(x, lengths):
    ml_bs = _argmax_call(x)
    tokens, counts2d = _sc_decode(ml_bs, lengths.astype(jnp.int32))
    return tokens, counts2d[:, 0]


# speedup vs baseline: 1.0284x; 1.0284x over previous
import functools

import jax
import jax.numpy as jnp
from jax import lax
from jax.experimental import pallas as pl
from jax.experimental.pallas import tpu as pltpu
from jax.experimental.pallas import tpu_sc as plsc

_BLANK = 0
_S, _B, _V = 2048, 16, 1024
_L = 16


_BS = 256
_NB = _S // _BS


def _argmax_body(x_ref, o_ref):
    xb = x_ref[...]
    m = jnp.max(xb, axis=2, keepdims=True)
    lane = lax.broadcasted_iota(jnp.int32, xb.shape, 2)
    idx = jnp.min(jnp.where(xb == m, lane, _V), axis=2)
    o_ref[...] = idx.astype(jnp.int32).T


_argmax_call = pl.pallas_call(
    _argmax_body,
    grid=(_NB,),
    in_specs=[pl.BlockSpec((_BS, _B, _V), lambda i: (i, 0, 0))],
    out_specs=pl.BlockSpec((_B, _BS), lambda i: (0, i)),
    out_shape=jax.ShapeDtypeStruct((_B, _S), jnp.int32),
)


@functools.partial(
    pl.kernel,
    out_type=[
        jax.ShapeDtypeStruct((_B, _S), jnp.int32),
        jax.ShapeDtypeStruct((_B, _L), jnp.int32),
    ],
    mesh=plsc.VectorSubcoreMesh(
        core_axis_name="c", subcore_axis_name="s", num_cores=1),
    compiler_params=pltpu.CompilerParams(needs_layout_passes=False),
    scratch_types=[
        pltpu.VMEM((_S,), jnp.int32),
        pltpu.VMEM((_L,), jnp.int32),
        pltpu.VMEM((_S,), jnp.int32),
        pltpu.VMEM((_L,), jnp.int32),
    ],
)
def _sc_decode(ml_hbm, len_hbm, tok_hbm, cnt_hbm,
               buf_v, len_v, out_v, cnt_v):
    c = lax.axis_index("c")
    s = lax.axis_index("s")

    @pl.when(c == 0)
    def _():
        iota = lax.iota(jnp.int32, _L)
        zero_v = jnp.zeros((_L,), jnp.int32)
        neg1 = jnp.full((_L,), -1, jnp.int32)

        pltpu.sync_copy(ml_hbm.at[s], buf_v)
        pltpu.sync_copy(len_hbm, len_v)
        length = jnp.sum(jnp.where(iota == s, len_v[...], 0))

        @plsc.parallel_loop(0, _S // _L, 1, unroll=8)
        def _init_body(i):
            out_v[pl.ds(i * _L, _L)] = neg1

        def step(base, off_vec):
            pos = base + iota
            v = buf_v[pl.ds(base, _L)]
            pgather = plsc.load_gather(buf_v, [jnp.maximum(pos - 1, 0)])
            prev = jnp.where(pos == 0, -1, pgather)
            valid = pos < length
            keep = (v != _BLANK) & (v != prev) & valid
            dest = off_vec + plsc.cumsum(keep.astype(jnp.int32)) - 1
            plsc.store_scatter(out_v, [dest], v, mask=keep)
            return off_vec + plsc.all_reduce_population_count(keep)

        nchunks = (length + _L - 1) // _L

        @plsc.parallel_loop(0, nchunks, 1, unroll=2, carry=zero_v)
        def total_vec(i, off_vec):
            return step(i * _L, off_vec)
        pltpu.sync_copy(out_v, tok_hbm.at[s])

        cnt_v[...] = total_vec
        pltpu.sync_copy(cnt_v, cnt_hbm.at[s])


def kernel(x, lengths):
    ml_bs = _argmax_call(x)
    tokens, counts2d = _sc_decode(ml_bs, lengths.astype(jnp.int32))
    return tokens, counts2d[:, 0]


# speedup vs baseline: 1.0304x; 1.0019x over previous
import functools

import jax
import jax.numpy as jnp
from jax import lax
from jax.experimental import pallas as pl
from jax.experimental.pallas import tpu as pltpu
from jax.experimental.pallas import tpu_sc as plsc

_BLANK = 0
_S, _B, _V = 2048, 16, 1024
_L = 16


_BS = 256
_NB = _S // _BS


def _argmax_body(x_ref, o_ref):
    xb = x_ref[...]
    m = jnp.max(xb, axis=2, keepdims=True)
    lane = lax.broadcasted_iota(jnp.int32, xb.shape, 2)
    idx = jnp.min(jnp.where(xb == m, lane, _V), axis=2)
    o_ref[...] = idx.astype(jnp.int32).T


_argmax_call = pl.pallas_call(
    _argmax_body,
    grid=(_NB,),
    in_specs=[pl.BlockSpec((_BS, _B, _V), lambda i: (i, 0, 0))],
    out_specs=pl.BlockSpec((_B, _BS), lambda i: (0, i)),
    out_shape=jax.ShapeDtypeStruct((_B, _S), jnp.int32),
)


@functools.partial(
    pl.kernel,
    out_type=[
        jax.ShapeDtypeStruct((_B, _S), jnp.int32),
        jax.ShapeDtypeStruct((_B, _L), jnp.int32),
    ],
    mesh=plsc.VectorSubcoreMesh(
        core_axis_name="c", subcore_axis_name="s", num_cores=1),
    compiler_params=pltpu.CompilerParams(needs_layout_passes=False),
    scratch_types=[
        pltpu.VMEM((_S,), jnp.int32),
        pltpu.VMEM((_L,), jnp.int32),
        pltpu.VMEM((_S,), jnp.int32),
        pltpu.VMEM((_L,), jnp.int32),
    ],
)
def _sc_decode(ml_hbm, len_hbm, tok_hbm, cnt_hbm,
               buf_v, len_v, out_v, cnt_v):
    c = lax.axis_index("c")
    s = lax.axis_index("s")

    @pl.when(c == 0)
    def _():
        iota = lax.iota(jnp.int32, _L)
        zero_v = jnp.zeros((_L,), jnp.int32)
        neg1 = jnp.full((_L,), -1, jnp.int32)

        pltpu.sync_copy(ml_hbm.at[s], buf_v)
        pltpu.sync_copy(len_hbm, len_v)
        length = jnp.sum(jnp.where(iota == s, len_v[...], 0))

        @plsc.parallel_loop(0, _S // _L, 1, unroll=8)
        def _init_body(i):
            out_v[pl.ds(i * _L, _L)] = neg1

        def step(base, off_vec):
            pos = base + iota
            v = buf_v[pl.ds(base, _L)]
            pgather = plsc.load_gather(buf_v, [jnp.maximum(pos - 1, 0)])
            prev = jnp.where(pos == 0, -1, pgather)
            valid = pos < length
            keep = (v != _BLANK) & (v != prev) & valid
            dest = off_vec + plsc.cumsum(keep.astype(jnp.int32)) - 1
            plsc.store_scatter(out_v, [dest], v, mask=keep)
            return off_vec + plsc.all_reduce_population_count(keep)

        nchunks = (length + _L - 1) // _L

        @plsc.parallel_loop(0, nchunks, 1, unroll=4, carry=zero_v)
        def total_vec(i, off_vec):
            return step(i * _L, off_vec)
        pltpu.sync_copy(out_v, tok_hbm.at[s])

        cnt_v[...] = total_vec
        pltpu.sync_copy(cnt_v, cnt_hbm.at[s])


def kernel(x, lengths):
    ml_bs = _argmax_call(x)
    tokens, counts2d = _sc_decode(ml_bs, lengths.astype(jnp.int32))
    return tokens, counts2d[:, 0]


# speedup vs baseline: 1.0370x; 1.0064x over previous
import functools

import jax
import jax.numpy as jnp
from jax import lax
from jax.experimental import pallas as pl
from jax.experimental.pallas import tpu as pltpu
from jax.experimental.pallas import tpu_sc as plsc

_BLANK = 0
_S, _B, _V = 2048, 16, 1024
_L = 16


_BS = 256
_NB = _S // _BS


def _argmax_body(x_ref, o_ref):
    xb = x_ref[...]
    m = jnp.max(xb, axis=2, keepdims=True)
    lane = lax.broadcasted_iota(jnp.int32, xb.shape, 2)
    idx = jnp.min(jnp.where(xb == m, lane, _V), axis=2)
    o_ref[...] = idx.astype(jnp.int32).T


_argmax_call = pl.pallas_call(
    _argmax_body,
    grid=(_NB,),
    in_specs=[pl.BlockSpec((_BS, _B, _V), lambda i: (i, 0, 0))],
    out_specs=pl.BlockSpec((_B, _BS), lambda i: (0, i)),
    out_shape=jax.ShapeDtypeStruct((_B, _S), jnp.int32),
)


@functools.partial(
    pl.kernel,
    out_type=[
        jax.ShapeDtypeStruct((_B, _S), jnp.int32),
        jax.ShapeDtypeStruct((_B, _L), jnp.int32),
        jax.ShapeDtypeStruct((_B,), jnp.int32),
    ],
    mesh=plsc.VectorSubcoreMesh(
        core_axis_name="c", subcore_axis_name="s", num_cores=1),
    compiler_params=pltpu.CompilerParams(needs_layout_passes=False),
    scratch_types=[
        pltpu.VMEM((_S,), jnp.int32),
        pltpu.VMEM((_L,), jnp.int32),
        pltpu.VMEM((_S,), jnp.int32),
        pltpu.VMEM((_L,), jnp.int32),
        pltpu.VMEM((_L, _L), jnp.int32),
    ],
)
def _sc_decode(ml_hbm, len_hbm, tok_hbm, cnt2d_hbm, cnt_hbm,
               buf_v, len_v, out_v, cnt_v, cnt_mat_v):
    c = lax.axis_index("c")
    s = lax.axis_index("s")

    @pl.when(c == 0)
    def _():
        iota = lax.iota(jnp.int32, _L)
        zero_v = jnp.zeros((_L,), jnp.int32)
        neg1 = jnp.full((_L,), -1, jnp.int32)

        pltpu.sync_copy(ml_hbm.at[s], buf_v)
        pltpu.sync_copy(len_hbm, len_v)
        length = jnp.sum(jnp.where(iota == s, len_v[...], 0))

        @plsc.parallel_loop(0, _S // _L, 1, unroll=8)
        def _init_body(i):
            out_v[pl.ds(i * _L, _L)] = neg1

        def step(base, off_vec):
            pos = base + iota
            v = buf_v[pl.ds(base, _L)]
            pgather = plsc.load_gather(buf_v, [jnp.maximum(pos - 1, 0)])
            prev = jnp.where(pos == 0, -1, pgather)
            valid = pos < length
            keep = (v != _BLANK) & (v != prev) & valid
            dest = off_vec + plsc.cumsum(keep.astype(jnp.int32)) - 1
            plsc.store_scatter(out_v, [dest], v, mask=keep)
            return off_vec + plsc.all_reduce_population_count(keep)

        nchunks = (length + _L - 1) // _L

        @plsc.parallel_loop(0, nchunks, 1, unroll=4, carry=zero_v)
        def total_vec(i, off_vec):
            return step(i * _L, off_vec)
        pltpu.sync_copy(out_v, tok_hbm.at[s])

        cnt_v[...] = total_vec
        pltpu.sync_copy(cnt_v, cnt2d_hbm.at[s])
        plsc.subcore_barrier()

        @pl.when(s == 0)
        def _():
            pltpu.sync_copy(cnt2d_hbm, cnt_mat_v)
            cnt_v[...] = plsc.load_gather(cnt_mat_v, [iota, iota])
            pltpu.sync_copy(cnt_v, cnt_hbm)


def kernel(x, lengths):
    ml_bs = _argmax_call(x)
    tokens, _staging, counts = _sc_decode(ml_bs, lengths.astype(jnp.int32))
    return tokens, counts
